# baseline (speedup 1.0000x reference)
import functools

import jax
import jax.numpy as jnp
from jax import lax
from jax.experimental import pallas as pl
from jax.experimental.pallas import tpu as pltpu
from jax.experimental.pallas import tpu_sc as plsc

N_OBJS = 1000000
D_OBJ = 64
N_ART = 10
D_ART = 32
BATCH = 16384

NC = 2
NS = 16
NW = NC * NS
B_PER_W = BATCH // NW
CHUNK = 128
N_CHUNKS = B_PER_W // CHUNK

_mesh = plsc.VectorSubcoreMesh(core_axis_name="c", subcore_axis_name="s")


@jax.jit
def _lookup(instance_id, articulation_id, W_shape, W_app, W_art):
    inst2d = instance_id.reshape(NW * N_CHUNKS, CHUNK)
    art2d = articulation_id.reshape(NW * N_CHUNKS, CHUNK)

    @functools.partial(
        pl.kernel,
        out_type=(
            jax.ShapeDtypeStruct((BATCH, D_OBJ), jnp.float32),
            jax.ShapeDtypeStruct((BATCH, D_OBJ), jnp.float32),
            jax.ShapeDtypeStruct((BATCH, D_ART), jnp.float32),
        ),
        mesh=_mesh,
        compiler_params=pltpu.CompilerParams(use_tc_tiling_on_sc=False),
        scratch_types=[
            pltpu.VMEM((N_CHUNKS, CHUNK), jnp.int32),
            pltpu.VMEM((N_CHUNKS, CHUNK), jnp.int32),
            pltpu.VMEM((B_PER_W, D_OBJ), jnp.float32),
            pltpu.VMEM((B_PER_W, D_OBJ), jnp.float32),
            pltpu.VMEM((B_PER_W, D_ART), jnp.float32),
            pltpu.SemaphoreType.DMA,
        ],
    )
    def k(ws_hbm, wa_hbm, wr_hbm, ii_hbm, ai_hbm,
          dens_hbm, col_hbm, art_hbm,
          ii_v, ai_v, dens_v, col_v, art_v, sem):
        wid = lax.axis_index("s") * NC + lax.axis_index("c")
        row0 = wid * N_CHUNKS
        base = wid * B_PER_W

        pltpu.sync_copy(ii_hbm.at[pl.ds(row0, N_CHUNKS)], ii_v)
        pltpu.sync_copy(ai_hbm.at[pl.ds(row0, N_CHUNKS)], ai_v)

        for j in range(N_CHUNKS):
            sl = pl.ds(j * CHUNK, CHUNK)
            pltpu.async_copy(ws_hbm.at[ii_v.at[j]], dens_v.at[sl], sem)
            pltpu.async_copy(wa_hbm.at[ii_v.at[j]], col_v.at[sl], sem)
            pltpu.async_copy(wr_hbm.at[ai_v.at[j]], art_v.at[sl], sem)
        for j in range(N_CHUNKS):
            sl = pl.ds(j * CHUNK, CHUNK)
            pltpu.make_async_copy(ws_hbm.at[ii_v.at[j]], dens_v.at[sl], sem).wait()
            pltpu.make_async_copy(wa_hbm.at[ii_v.at[j]], col_v.at[sl], sem).wait()
            pltpu.make_async_copy(wr_hbm.at[ai_v.at[j]], art_v.at[sl], sem).wait()

        pltpu.sync_copy(dens_v, dens_hbm.at[pl.ds(base, B_PER_W)])
        pltpu.sync_copy(col_v, col_hbm.at[pl.ds(base, B_PER_W)])
        pltpu.sync_copy(art_v, art_hbm.at[pl.ds(base, B_PER_W)])

    return k(W_shape, W_app, W_art, inst2d, art2d)


def kernel(instance_id, articulation_id, W_shape, W_app, W_art):
    return _lookup(
        instance_id.astype(jnp.int32),
        articulation_id.astype(jnp.int32),
        W_shape,
        W_app,
        W_art,
    )
